# baseline (speedup 1.0000x reference)
import functools

import jax
import jax.numpy as jnp
from jax import lax
from jax.experimental import pallas as pl
from jax.experimental.pallas import tpu as pltpu
from jax.experimental.pallas import tpu_sc as plsc

DIM = 256
BIG_I32 = 2**30


def _dist_argmin_body(n_code_blocks, code_block, x_ref, e_ref, ind_ref,
                      run_min, run_idx):
    nc = pl.program_id(1)
    x = x_ref[...]
    e = e_ref[...]
    c = jnp.sum(e * e, axis=0, keepdims=True)
    b = jnp.dot(x, -2.0 * e, preferred_element_type=jnp.float32)
    d = b + c
    if n_code_blocks == 1:
        ind_ref[...] = jnp.argmin(d, axis=1, keepdims=True).astype(jnp.int32)
        return
    m = jnp.min(d, axis=1, keepdims=True)
    iota = lax.broadcasted_iota(jnp.int32, d.shape, 1) + nc * code_block
    lidx = jnp.min(jnp.where(d == m, iota, BIG_I32), axis=1, keepdims=True)

    @pl.when(nc == 0)
    def _():
        run_min[...] = m
        run_idx[...] = lidx

    @pl.when(nc > 0)
    def _():
        better = m < run_min[...]
        run_idx[...] = jnp.where(better, lidx, run_idx[...])
        run_min[...] = jnp.where(better, m, run_min[...])

    @pl.when(nc == n_code_blocks - 1)
    def _():
        ind_ref[...] = run_idx[...]


def _argmin_call(x, embed, token_block, code_block):
    n_tok, dim = x.shape
    n_embed = embed.shape[1]
    nt, ncb = n_tok // token_block, n_embed // code_block
    return pl.pallas_call(
        functools.partial(_dist_argmin_body, ncb, code_block),
        grid=(nt, ncb),
        in_specs=[
            pl.BlockSpec((token_block, dim), lambda i, j: (i, 0)),
            pl.BlockSpec((dim, code_block), lambda i, j: (0, j)),
        ],
        out_specs=pl.BlockSpec((token_block, 1), lambda i, j: (i, 0)),
        out_shape=jax.ShapeDtypeStruct((n_tok, 1), jnp.int32),
        scratch_shapes=[
            pltpu.VMEM((token_block, 1), jnp.float32),
            pltpu.VMEM((token_block, 1), jnp.int32),
        ],
    )(x, embed)


def _sc_gather_st(table, idx, x):
    v, d = table.shape
    b = idx.shape[0]
    info = plsc.get_sparse_core_info()
    nw = info.num_cores * info.num_subcores
    b_per_w = b // nw
    chunk = 64
    n_chunks = b_per_w // chunk
    n_lane = d // 16
    mesh = plsc.VectorSubcoreMesh(core_axis_name="c", subcore_axis_name="s")

    @functools.partial(
        pl.kernel,
        mesh=mesh,
        out_type=[
            jax.ShapeDtypeStruct((b, d), jnp.float32),
            jax.ShapeDtypeStruct((nw, 16), jnp.float32),
        ],
        scratch_types=[
            pltpu.VMEM((b_per_w,), jnp.int32),
            pltpu.VMEM((b_per_w, d), jnp.float32),
            pltpu.VMEM((chunk, d), jnp.float32),
            pltpu.VMEM((16,), jnp.float32),
            pltpu.SemaphoreType.DMA,
        ],
    )
    def gather_kernel(table_hbm, idx_hbm, x_hbm, out_hbm, part_hbm,
                      idx_v, rows_v, x_v, acc_v, sem):
        wid = lax.axis_index("s") * info.num_cores + lax.axis_index("c")
        base = wid * b_per_w
        pltpu.sync_copy(idx_hbm.at[pl.ds(base, b_per_w)], idx_v)
        pltpu.async_copy(table_hbm.at[idx_v], rows_v, sem).wait()
        acc_v[...] = jnp.zeros((16,), jnp.float32)

        def chunk_body(c, _):
            pltpu.sync_copy(x_hbm.at[pl.ds(base + c * chunk, chunk)], x_v)

            def row_body(r, _):
                for l in range(n_lane):
                    sl = pl.ds(l * 16, 16)
                    q = rows_v[c * chunk + r, sl]
                    xv = x_v[r, sl]
                    t = q - xv
                    rows_v[c * chunk + r, sl] = xv + t
                    acc_v[...] += t * t
                return 0

            lax.fori_loop(0, chunk, row_body, 0)
            pltpu.sync_copy(
                rows_v.at[pl.ds(c * chunk, chunk)],
                out_hbm.at[pl.ds(base + c * chunk, chunk)])
            return 0

        lax.fori_loop(0, n_chunks, chunk_body, 0)
        pltpu.sync_copy(acc_v, part_hbm.at[wid])

    return gather_kernel(table, idx, x)


def _epilogue_body(n_blocks, inv_total, x_ref, q_ref, qst_ref, diff_ref, acc):
    i = pl.program_id(0)
    x = x_ref[...]
    q = q_ref[...]
    qst_ref[...] = x + (q - x)
    s = jnp.sum((q - x) ** 2)

    @pl.when(i == 0)
    def _():
        acc[0] = 0.0

    acc[0] += s

    @pl.when(i == n_blocks - 1)
    def _():
        diff_ref[...] = jnp.broadcast_to(acc[0] * inv_total, (1, 1))


def _epilogue_call(x, q, token_block):
    n_tok, dim = x.shape
    nb = n_tok // token_block
    inv_total = 1.0 / float(n_tok * dim)
    return pl.pallas_call(
        functools.partial(_epilogue_body, nb, inv_total),
        grid=(nb,),
        in_specs=[
            pl.BlockSpec((token_block, dim), lambda i: (i, 0)),
            pl.BlockSpec((token_block, dim), lambda i: (i, 0)),
        ],
        out_specs=[
            pl.BlockSpec((token_block, dim), lambda i: (i, 0)),
            pl.BlockSpec((1, 1), lambda i: (0, 0)),
        ],
        out_shape=[
            jax.ShapeDtypeStruct((n_tok, dim), jnp.float32),
            jax.ShapeDtypeStruct((1, 1), jnp.float32),
        ],
        scratch_shapes=[pltpu.SMEM((1,), jnp.float32)],
    )(x, q)


def kernel(input, embed):
    n_tok = input.size // DIM
    x = input.reshape(n_tok, DIM)
    ind2 = _argmin_call(x, embed, token_block=1024, code_block=8192)
    ind = ind2.reshape(n_tok)
    et = embed.T
    qst, partials = _sc_gather_st(et, ind, x)
    diff = jnp.sum(partials) * (1.0 / float(n_tok * DIM))
    quantize_st = qst.reshape(input.shape)
    embed_ind = ind.reshape(input.shape[:-1])
    return quantize_st, diff, jnp.zeros(1, dtype=jnp.float32), embed_ind


# speedup vs baseline: 1.3882x; 1.3882x over previous
import functools

import jax
import jax.numpy as jnp
from jax import lax
from jax.experimental import pallas as pl
from jax.experimental.pallas import tpu as pltpu
from jax.experimental.pallas import tpu_sc as plsc

DIM = 256
BIG_I32 = 2**30


def _dist_argmin_body(n_code_blocks, code_block, x_ref, e_ref, ind_ref,
                      run_min, run_idx):
    nc = pl.program_id(1)
    x = x_ref[...]
    e = e_ref[...]
    c = jnp.sum(e * e, axis=0, keepdims=True)
    b = jnp.dot(x, -2.0 * e, preferred_element_type=jnp.float32)
    d = b + c
    if n_code_blocks == 1:
        ind_ref[...] = jnp.argmin(d, axis=1, keepdims=True).astype(jnp.int32)
        return
    m = jnp.min(d, axis=1, keepdims=True)
    iota = lax.broadcasted_iota(jnp.int32, d.shape, 1) + nc * code_block
    lidx = jnp.min(jnp.where(d == m, iota, BIG_I32), axis=1, keepdims=True)

    @pl.when(nc == 0)
    def _():
        run_min[...] = m
        run_idx[...] = lidx

    @pl.when(nc > 0)
    def _():
        better = m < run_min[...]
        run_idx[...] = jnp.where(better, lidx, run_idx[...])
        run_min[...] = jnp.where(better, m, run_min[...])

    @pl.when(nc == n_code_blocks - 1)
    def _():
        ind_ref[...] = run_idx[...]


def _argmin_call(x, embed, token_block, code_block):
    n_tok, dim = x.shape
    n_embed = embed.shape[1]
    nt, ncb = n_tok // token_block, n_embed // code_block
    return pl.pallas_call(
        functools.partial(_dist_argmin_body, ncb, code_block),
        grid=(nt, ncb),
        in_specs=[
            pl.BlockSpec((token_block, dim), lambda i, j: (i, 0)),
            pl.BlockSpec((dim, code_block), lambda i, j: (0, j)),
        ],
        out_specs=pl.BlockSpec((token_block, 1), lambda i, j: (i, 0)),
        out_shape=jax.ShapeDtypeStruct((n_tok, 1), jnp.int32),
        scratch_shapes=[
            pltpu.VMEM((token_block, 1), jnp.float32),
            pltpu.VMEM((token_block, 1), jnp.int32),
        ],
    )(x, embed)


def _sc_gather(table, idx):
    v, d = table.shape
    b = idx.shape[0]
    info = plsc.get_sparse_core_info()
    nw = info.num_cores * info.num_subcores
    b_per_w = b // nw
    mesh = plsc.VectorSubcoreMesh(core_axis_name="c", subcore_axis_name="s")

    @functools.partial(
        pl.kernel,
        mesh=mesh,
        out_type=jax.ShapeDtypeStruct((b, d), jnp.float32),
        scratch_types=[
            pltpu.VMEM((b_per_w,), jnp.int32),
            pltpu.VMEM((b_per_w, d), jnp.float32),
            pltpu.SemaphoreType.DMA,
        ],
    )
    def gather_kernel(table_hbm, idx_hbm, out_hbm, idx_v, rows_v, sem):
        wid = lax.axis_index("s") * info.num_cores + lax.axis_index("c")
        base = wid * b_per_w
        pltpu.sync_copy(idx_hbm.at[pl.ds(base, b_per_w)], idx_v)
        pltpu.async_copy(table_hbm.at[idx_v], rows_v, sem).wait()
        pltpu.sync_copy(rows_v, out_hbm.at[pl.ds(base, b_per_w)])

    return gather_kernel(table, idx)


def _epilogue_body(n_blocks, inv_total, x_ref, q_ref, qst_ref, diff_ref, acc):
    i = pl.program_id(0)
    x = x_ref[...]
    q = q_ref[...]
    qst_ref[...] = x + (q - x)
    s = jnp.sum((q - x) ** 2)

    @pl.when(i == 0)
    def _():
        acc[0] = 0.0

    acc[0] += s

    @pl.when(i == n_blocks - 1)
    def _():
        diff_ref[...] = jnp.broadcast_to(acc[0] * inv_total, (1, 1))


def _epilogue_call(x, q, token_block):
    n_tok, dim = x.shape
    nb = n_tok // token_block
    inv_total = 1.0 / float(n_tok * dim)
    return pl.pallas_call(
        functools.partial(_epilogue_body, nb, inv_total),
        grid=(nb,),
        in_specs=[
            pl.BlockSpec((token_block, dim), lambda i: (i, 0)),
            pl.BlockSpec((token_block, dim), lambda i: (i, 0)),
        ],
        out_specs=[
            pl.BlockSpec((token_block, dim), lambda i: (i, 0)),
            pl.BlockSpec((1, 1), lambda i: (0, 0)),
        ],
        out_shape=[
            jax.ShapeDtypeStruct((n_tok, dim), jnp.float32),
            jax.ShapeDtypeStruct((1, 1), jnp.float32),
        ],
        scratch_shapes=[pltpu.SMEM((1,), jnp.float32)],
    )(x, q)


def kernel(input, embed):
    n_tok = input.size // DIM
    x = input.reshape(n_tok, DIM)
    ind2 = _argmin_call(x, embed, token_block=1024, code_block=8192)
    ind = ind2.reshape(n_tok)
    et = embed.T
    q = _sc_gather(et, ind)
    qst, diff = _epilogue_call(x, q, token_block=1024)
    quantize_st = qst.reshape(input.shape)
    embed_ind = ind.reshape(input.shape[:-1])
    return quantize_st, diff[0, 0], jnp.zeros(1, dtype=jnp.float32), embed_ind
